# baseline (speedup 1.0000x reference)
import jax
import jax.numpy as jnp
from jax import lax
from jax.experimental import pallas as pl
from jax.experimental.pallas import tpu as pltpu
from jax.experimental.pallas import tpu_sc as plsc

VOCAB = 100000
D_MODEL = 128
MAX_LEN = 512
BATCH = 4096
SEQ = 200

NUM_CORES = 2
NUM_SUBCORES = 16
NW = NUM_CORES * NUM_SUBCORES
ROWS_PER_W = BATCH * SEQ // NW
LANES = 16

SLOT = 128
NSLOT = ROWS_PER_W // SLOT
NBUF = 4
AHEAD = 1
NI = NSLOT // NBUF


def _sine_pe():
    pos = jnp.arange(MAX_LEN, dtype=jnp.float32)[:, None]
    div = jnp.exp(
        jnp.arange(0, D_MODEL, 2, dtype=jnp.float32)
        * (-jnp.log(10000.0) / D_MODEL)
    )
    pe = jnp.zeros((MAX_LEN, D_MODEL), dtype=jnp.float32)
    pe = pe.at[:, 0::2].set(jnp.sin(pos * div))
    pe = pe.at[:, 1::2].set(jnp.cos(pos * div))
    return pe[:SEQ]


def _body(ids, table, pe, out, idx_v, pe_v, *bufs):
    rows = bufs[:NBUF]
    gs = bufs[NBUF:2 * NBUF]
    os_ = bufs[2 * NBUF:3 * NBUF]

    c = lax.axis_index("c")
    s = lax.axis_index("s")
    wid = s * NUM_CORES + c
    base = wid * ROWS_PER_W

    pltpu.sync_copy(pe, pe_v)
    pltpu.sync_copy(ids.at[pl.ds(base, ROWS_PER_W)], idx_v)

    def gather_copy(slot, b):
        return pltpu.make_async_copy(
            table.at[idx_v.at[pl.ds(slot * SLOT, SLOT)]],
            rows[b],
            gs[b],
        )

    def out_copy(slot, b):
        return pltpu.make_async_copy(
            rows[b],
            out.at[pl.ds(base + slot * SLOT, SLOT)],
            os_[b],
        )

    def add_pe(slot, b):
        pe_off = lax.rem(slot * SLOT, SEQ)
        k = jnp.minimum(SEQ - pe_off, SLOT)

        def make_body(delta):
            def row_body(r, carry):
                for cc in range(D_MODEL // LANES):
                    plsc.addupdate(
                        rows[b].at[r, pl.ds(cc * LANES, LANES)],
                        pe_v[pe_off + r + delta, pl.ds(cc * LANES, LANES)],
                    )
                return carry
            return row_body

        lax.fori_loop(0, k, make_body(0), 0)
        lax.fori_loop(k, SLOT, make_body(-SEQ), 0)

    for b in range(AHEAD):
        gather_copy(b, b).start()

    def pipe_body(i, carry):
        for b in range(NBUF):
            slot = i * NBUF + b
            gather_copy(slot, b).wait()
            nb = (b + AHEAD) % NBUF

            @pl.when(slot + AHEAD >= NBUF)
            def _():
                out_copy(slot + AHEAD - NBUF, nb).wait()

            @pl.when(slot + AHEAD < NSLOT)
            def _():
                gather_copy(slot + AHEAD, nb).start()

            add_pe(slot, b)
            out_copy(slot, b).start()
        return carry

    lax.fori_loop(0, NI, pipe_body, 0)
    for k in range(NBUF - AHEAD, 0, -1):
        slot = NSLOT - k
        out_copy(slot, slot % NBUF).wait()


def kernel(input_ids, emb_table):
    batch, seq = input_ids.shape
    _, d = emb_table.shape
    pe = _sine_pe()
    fn = pl.kernel(
        _body,
        out_type=jax.ShapeDtypeStruct((batch * seq, d), jnp.float32),
        mesh=plsc.VectorSubcoreMesh(
            core_axis_name="c", subcore_axis_name="s"
        ),
        compiler_params=pltpu.CompilerParams(use_tc_tiling_on_sc=False),
        scratch_types=(
            [pltpu.VMEM((ROWS_PER_W,), jnp.int32)]
            + [pltpu.VMEM((SEQ, D_MODEL), jnp.float32)]
            + [pltpu.VMEM((SLOT, D_MODEL), jnp.float32)] * NBUF
            + [pltpu.SemaphoreType.DMA] * (2 * NBUF)
        ),
    )
    flat = fn(input_ids.reshape(-1).astype(jnp.int32), emb_table, pe)
    return flat.reshape(batch, seq, d)


# speedup vs baseline: 2.0732x; 2.0732x over previous
import jax
import jax.numpy as jnp
from jax import lax
from jax.experimental import pallas as pl
from jax.experimental.pallas import tpu as pltpu
from jax.experimental.pallas import tpu_sc as plsc

VOCAB = 100000
D_MODEL = 128
MAX_LEN = 512
BATCH = 4096
SEQ = 200

NUM_CORES = 2
NUM_SUBCORES = 16
NW = NUM_CORES * NUM_SUBCORES
ROWS_PER_W = BATCH * SEQ // NW
LANES = 16
SLOTS = ((0, 104, 0), (104, 96, 104), (200, 104, 0), (304, 96, 104))
NBUF = 4
NP = ROWS_PER_W // 400


def _sine_pe():
    pos = jnp.arange(MAX_LEN, dtype=jnp.float32)[:, None]
    div = jnp.exp(
        jnp.arange(0, D_MODEL, 2, dtype=jnp.float32)
        * (-jnp.log(10000.0) / D_MODEL)
    )
    pe = jnp.zeros((MAX_LEN, D_MODEL), dtype=jnp.float32)
    pe = pe.at[:, 0::2].set(jnp.sin(pos * div))
    pe = pe.at[:, 1::2].set(jnp.cos(pos * div))
    return pe[:SEQ]


def _body(ids, table, pe, out, idx_v, pe_v, r0, r1, r2, r3,
          g0, g1, g2, g3, o0, o1, o2, o3):
    c = lax.axis_index("c")
    s = lax.axis_index("s")
    wid = s * NUM_CORES + c
    base = wid * ROWS_PER_W

    rows = (r0, r1, r2, r3)
    gs = (g0, g1, g2, g3)
    os_ = (o0, o1, o2, o3)

    pltpu.sync_copy(pe, pe_v)
    pltpu.sync_copy(ids.at[pl.ds(base, ROWS_PER_W)], idx_v)

    def gather_copy(p, b):
        off, ln, _ = SLOTS[b]
        lo = p * 400 + off
        return pltpu.make_async_copy(
            table.at[idx_v.at[pl.ds(lo, ln)]],
            rows[b].at[pl.ds(0, ln)],
            gs[b],
        )

    def out_copy(p, b):
        off, ln, _ = SLOTS[b]
        lo = p * 400 + off
        return pltpu.make_async_copy(
            rows[b].at[pl.ds(0, ln)],
            out.at[pl.ds(base + lo, ln)],
            os_[b],
        )

    def add_pe(p, b):
        _, ln, pe_off = SLOTS[b]

        @plsc.parallel_loop(0, ln, unroll=4)
        def row_body(r):
            for cc in range(D_MODEL // LANES):
                plsc.addupdate(
                    rows[b].at[r, pl.ds(cc * LANES, LANES)],
                    pe_v[pe_off + r, pl.ds(cc * LANES, LANES)],
                )

    gather_copy(0, 0).start()

    def pipe_body(p, carry):
        for b in range(NBUF):
            gather_copy(p, b).wait()
            if b < NBUF - 1:
                @pl.when(p >= 1)
                def _():
                    out_copy(p - 1, b + 1).wait()

                gather_copy(p, b + 1).start()
            else:
                @pl.when(p + 1 < NP)
                def _():
                    out_copy(p, 0).wait()
                    gather_copy(p + 1, 0).start()

            add_pe(p, b)
            out_copy(p, b).start()
        return carry

    lax.fori_loop(0, NP, pipe_body, 0)
    for b in range(NBUF):
        out_copy(NP - 1, b).wait()


def kernel(input_ids, emb_table):
    batch, seq = input_ids.shape
    _, d = emb_table.shape
    pe = _sine_pe()
    fn = pl.kernel(
        _body,
        out_type=jax.ShapeDtypeStruct((batch * seq, d), jnp.float32),
        mesh=plsc.VectorSubcoreMesh(
            core_axis_name="c", subcore_axis_name="s"
        ),
        compiler_params=pltpu.CompilerParams(use_tc_tiling_on_sc=False),
        scratch_types=(
            [pltpu.VMEM((ROWS_PER_W,), jnp.int32)]
            + [pltpu.VMEM((SEQ, D_MODEL), jnp.float32)]
            + [pltpu.VMEM((104, D_MODEL), jnp.float32)] * NBUF
            + [pltpu.SemaphoreType.DMA] * (2 * NBUF)
        ),
    )
    flat = fn(input_ids.reshape(-1).astype(jnp.int32), emb_table, pe)
    return flat.reshape(batch, seq, d)


# speedup vs baseline: 2.6090x; 1.2584x over previous
import jax
import jax.numpy as jnp
from jax import lax
from jax.experimental import pallas as pl
from jax.experimental.pallas import tpu as pltpu
from jax.experimental.pallas import tpu_sc as plsc

VOCAB = 100000
D_MODEL = 128
MAX_LEN = 512
BATCH = 4096
SEQ = 200

NUM_CORES = 2
NUM_SUBCORES = 16
NW = NUM_CORES * NUM_SUBCORES
ROWS_PER_W = BATCH * SEQ // NW
LANES = 16
SLOTS = ((0, 104, 0), (104, 96, 104), (200, 104, 0), (304, 96, 104))
NBUF = 4
NP = ROWS_PER_W // 400


def _sine_pe():
    pos = jnp.arange(MAX_LEN, dtype=jnp.float32)[:, None]
    div = jnp.exp(
        jnp.arange(0, D_MODEL, 2, dtype=jnp.float32)
        * (-jnp.log(10000.0) / D_MODEL)
    )
    pe = jnp.zeros((MAX_LEN, D_MODEL), dtype=jnp.float32)
    pe = pe.at[:, 0::2].set(jnp.sin(pos * div))
    pe = pe.at[:, 1::2].set(jnp.cos(pos * div))
    return pe[:SEQ]


def _body(ids, table, pe, out, idx_v, pe_v, r0, r1, r2, r3,
          g0, g1, g2, g3, o0, o1, o2, o3):
    c = lax.axis_index("c")
    s = lax.axis_index("s")
    wid = s * NUM_CORES + c
    base = wid * ROWS_PER_W

    rows = (r0, r1, r2, r3)
    gs = (g0, g1, g2, g3)
    os_ = (o0, o1, o2, o3)

    pltpu.sync_copy(pe, pe_v)
    pltpu.sync_copy(ids.at[pl.ds(base, ROWS_PER_W)], idx_v)

    def gather_copy(p, b):
        off, ln, _ = SLOTS[b]
        lo = p * 400 + off
        return pltpu.make_async_copy(
            table.at[idx_v.at[pl.ds(lo, ln)]],
            rows[b].at[pl.ds(0, ln)],
            gs[b],
        )

    def out_copy(p, b):
        off, ln, _ = SLOTS[b]
        lo = p * 400 + off
        return pltpu.make_async_copy(
            rows[b].at[pl.ds(0, ln)],
            out.at[pl.ds(base + lo, ln)],
            os_[b],
        )

    def add_pe(p, b):
        _, ln, pe_off = SLOTS[b]

        @plsc.parallel_loop(0, ln, unroll=4)
        def row_body(r):
            for cc in range(D_MODEL // LANES):
                plsc.addupdate(
                    rows[b].at[r, pl.ds(cc * LANES, LANES)],
                    pe_v[pe_off + r, pl.ds(cc * LANES, LANES)],
                )

    gather_copy(0, 0).start()
    gather_copy(0, 1).start()

    def pipe_body(p, carry):
        for b in range(NBUF):
            nb = (b + 2) % NBUF
            p2 = p if b < 2 else p + 1

            gather_copy(p, b).wait()

            @pl.when(p2 < NP)
            def _():
                @pl.when(p2 >= 1)
                def _():
                    out_copy(p2 - 1, nb).wait()

                gather_copy(p2, nb).start()

            add_pe(p, b)
            out_copy(p, b).start()
        return carry

    lax.fori_loop(0, NP, pipe_body, 0)
    for b in range(NBUF):
        out_copy(NP - 1, b).wait()


def kernel(input_ids, emb_table):
    batch, seq = input_ids.shape
    _, d = emb_table.shape
    pe = _sine_pe()
    fn = pl.kernel(
        _body,
        out_type=jax.ShapeDtypeStruct((batch * seq, d), jnp.float32),
        mesh=plsc.VectorSubcoreMesh(
            core_axis_name="c", subcore_axis_name="s"
        ),
        compiler_params=pltpu.CompilerParams(use_tc_tiling_on_sc=False),
        scratch_types=(
            [pltpu.VMEM((ROWS_PER_W,), jnp.int32)]
            + [pltpu.VMEM((SEQ, D_MODEL), jnp.float32)]
            + [pltpu.VMEM((104, D_MODEL), jnp.float32)] * NBUF
            + [pltpu.SemaphoreType.DMA] * (2 * NBUF)
        ),
    )
    flat = fn(input_ids.reshape(-1).astype(jnp.int32), emb_table, pe)
    return flat.reshape(batch, seq, d)
